# baseline (speedup 1.0000x reference)
import functools

import jax
import jax.numpy as jnp
from jax import lax
from jax.experimental import pallas as pl
from jax.experimental.pallas import tpu as pltpu


def _fold421_rows(rows):
    a4 = [rows[s] + rows[s + 4] for s in range(4)]
    a2 = [a4[s] + a4[s + 2] for s in range(2)]
    return a2[0] + a2[1]


def _lane64_reduce(s):
    acc = s[:, 0:8]
    for k in range(1, 8):
        acc = acc + s[:, 8 * k:8 * k + 8]
    a = acc[:, 0:4] + acc[:, 4:8]
    a = a[:, 0:2] + a[:, 2:4]
    return a[:, 0:1] + a[:, 1:2]


def _score_body(ids_ref, xb_ref, xs_ref, pos_ref, ker_ref):
    xs = xs_ref[0]

    count = jnp.zeros((), jnp.float32)
    parts = []
    for w0 in range(0, 512, 128):
        accs = [None] * 8
        for j in range(w0, w0 + 128, 8):
            for s in range(8):
                idx = ids_ref[0, 0, j + s]
                msk = (idx >= 0)
                idxc = jnp.maximum(idx, 0)
                row = xb_ref[0, pl.ds(idxc, 1), :] * jnp.where(msk, 1.0, 0.0).astype(jnp.float32)
                count = count + jnp.where(msk, 1.0, 0.0).astype(jnp.float32)
                accs[s] = row if accs[s] is None else accs[s] + row
        parts.append(_fold421_rows(accs))
    ksum = parts[0]
    for p in parts[1:]:
        ksum = ksum + p
    denom = jnp.maximum(count, jnp.float32(1.0))
    kv = ksum / denom
    ker_ref[0] = kv

    kn2 = _lane64_reduce(kv * kv)
    k_norm = jnp.maximum(jnp.sqrt(kn2), jnp.float32(1e-8))

    xst = xs.T
    sq = xst * xst
    accn = sq[0:8]
    for kk in range(1, 8):
        accn = accn + sq[8 * kk:8 * kk + 8]
    a4 = accn[0:4] + accn[4:8]
    a2 = a4[0:2] + a4[2:4]
    xs_n2 = a2[0:1] + a2[1:2]
    xs_norm = jnp.maximum(jnp.sqrt(xs_n2), jnp.float32(1e-8))

    kpad = jnp.concatenate([kv, jnp.zeros((7, 64), jnp.float32)], axis=0)
    dg = lax.dot_general(kpad.astype(jnp.bfloat16), xst.astype(jnp.bfloat16),
                         (((1,), (0,)), ((), ())),
                         preferred_element_type=jnp.float32)
    dots = dg[0:1, :]

    cos = dots / (xs_norm * k_norm)
    pos = (cos + jnp.float32(1.0)) / jnp.float32(2.0)
    gate = (count > 0).astype(jnp.float32)
    pos_ref[0] = pos * gate


def _topk_body(pos_ref, val_ref, idx_ref):
    v0 = pos_ref[...]

    def _treemax(ts):
        while len(ts) > 1:
            ts = [jnp.maximum(ts[2 * i], ts[2 * i + 1]) for i in range(len(ts) // 2)]
        return ts[0]

    def _treemin(ts):
        while len(ts) > 1:
            ts = [jnp.minimum(ts[2 * i], ts[2 * i + 1]) for i in range(len(ts) // 2)]
        return ts[0]

    def _half_step(v, iota):
        t = _treemax([v[:, 128 * g:128 * (g + 1)] for g in range(16)])
        mx = jnp.max(t, axis=1, keepdims=True)
        cand = jnp.where(v == mx, iota, jnp.int32(2048))
        tc = _treemin([cand[:, 128 * g:128 * (g + 1)] for g in range(16)])
        am = jnp.min(tc, axis=1, keepdims=True)
        vn = jnp.where(iota == am, -jnp.inf, v)
        return vn, mx, am

    iota4 = lax.broadcasted_iota(jnp.int32, (4, 2048), 1)
    riota = lax.broadcasted_iota(jnp.int32, (16, 256), 1)

    def body(r, carry):
        vh, vals, idxs = carry
        outs = [_half_step(v, iota4) for v in vh]
        vh = tuple(o[0] for o in outs)
        mx = jnp.concatenate([o[1] for o in outs], axis=0)
        am = jnp.concatenate([o[2] for o in outs], axis=0)
        sel = riota == r
        vals = jnp.where(sel, mx, vals)
        idxs = jnp.where(sel, am, idxs)
        return vh, vals, idxs

    vals0 = jnp.zeros((16, 256), jnp.float32)
    idxs0 = jnp.zeros((16, 256), jnp.int32)
    _, vals, idxs = lax.fori_loop(
        0, 256, body,
        ((v0[0:4], v0[4:8], v0[8:12], v0[12:16]), vals0, idxs0))
    val_ref[...] = vals
    idx_ref[...] = idxs


def _onehot_body(idx_ref, val_ref, out_ref):
    idr = idx_ref[0, 0]
    valr = val_ref[0, 0]
    idc = idr.reshape(64, 1)
    vc = valr.reshape(64, 1)
    iota = lax.broadcasted_iota(jnp.int32, (64, 2048), 1)
    oh = (iota == idc) & (vc > 0)
    out_ref[0] = oh.astype(jnp.float32)


def kernel(x_b, x_s, base_idxs):
    B, N, C = x_b.shape
    m = base_idxs.shape[1] // 2
    k = N // 8

    ids3 = base_idxs[:, :m].reshape(B, 1, m)

    pos3, ker3 = pl.pallas_call(
        _score_body,
        grid=(B,),
        in_specs=[
            pl.BlockSpec((1, 1, m), lambda b: (b, 0, 0),
                         memory_space=pltpu.SMEM),
            pl.BlockSpec((1, N, C), lambda b: (b, 0, 0)),
            pl.BlockSpec((1, N, C), lambda b: (b, 0, 0)),
        ],
        out_specs=[
            pl.BlockSpec((1, 1, N), lambda b: (b, 0, 0)),
            pl.BlockSpec((1, 1, C), lambda b: (b, 0, 0)),
        ],
        out_shape=[
            jax.ShapeDtypeStruct((B, 1, N), jnp.float32),
            jax.ShapeDtypeStruct((B, 1, C), jnp.float32),
        ],
    )(ids3, x_b, x_s)
    pos_scores = pos3.reshape(B, N)
    kernels = ker3.reshape(B, C)

    topk_val, topk_idx = pl.pallas_call(
        _topk_body,
        out_shape=[
            jax.ShapeDtypeStruct((B, k), jnp.float32),
            jax.ShapeDtypeStruct((B, k), jnp.int32),
        ],
    )(pos_scores)

    idx4 = topk_idx.reshape(B, 4, 1, 64)
    val4 = topk_val.reshape(B, 4, 1, 64)
    selected = pl.pallas_call(
        _onehot_body,
        grid=(B, 4),
        in_specs=[
            pl.BlockSpec((1, 1, 1, 64), lambda b, j: (b, j, 0, 0)),
            pl.BlockSpec((1, 1, 1, 64), lambda b, j: (b, j, 0, 0)),
        ],
        out_specs=pl.BlockSpec((1, 64, N), lambda b, j: (b, j, 0)),
        out_shape=jax.ShapeDtypeStruct((B, k, N), jnp.float32),
    )(idx4, val4)

    return (selected, topk_idx, pos_scores, x_s, kernels[:, :, None])


# speedup vs baseline: 1.1834x; 1.1834x over previous
import functools

import jax
import jax.numpy as jnp
from jax import lax
from jax.experimental import pallas as pl
from jax.experimental.pallas import tpu as pltpu


def _fold421_rows(rows):
    a4 = [rows[s] + rows[s + 4] for s in range(4)]
    a2 = [a4[s] + a4[s + 2] for s in range(2)]
    return a2[0] + a2[1]


def _lane64_reduce(s):
    acc = s[:, 0:8]
    for k in range(1, 8):
        acc = acc + s[:, 8 * k:8 * k + 8]
    a = acc[:, 0:4] + acc[:, 4:8]
    a = a[:, 0:2] + a[:, 2:4]
    return a[:, 0:1] + a[:, 1:2]


def _score_body(ids_ref, xb_ref, xs_ref, pos_ref, ker_ref):
    xs = xs_ref[0]

    count = jnp.zeros((), jnp.float32)
    parts = []
    for w0 in range(0, 512, 128):
        accs = [None] * 8
        for j in range(w0, w0 + 128, 8):
            for s in range(8):
                idx = ids_ref[0, 0, j + s]
                msk = (idx >= 0)
                idxc = jnp.maximum(idx, 0)
                row = xb_ref[0, pl.ds(idxc, 1), :] * jnp.where(msk, 1.0, 0.0).astype(jnp.float32)
                count = count + jnp.where(msk, 1.0, 0.0).astype(jnp.float32)
                accs[s] = row if accs[s] is None else accs[s] + row
        parts.append(_fold421_rows(accs))
    ksum = parts[0]
    for p in parts[1:]:
        ksum = ksum + p
    denom = jnp.maximum(count, jnp.float32(1.0))
    kv = ksum / denom
    ker_ref[0] = kv

    kn2 = _lane64_reduce(kv * kv)
    k_norm = jnp.maximum(jnp.sqrt(kn2), jnp.float32(1e-8))

    xst = xs.T
    sq = xst * xst
    accn = sq[0:8]
    for kk in range(1, 8):
        accn = accn + sq[8 * kk:8 * kk + 8]
    a4 = accn[0:4] + accn[4:8]
    a2 = a4[0:2] + a4[2:4]
    xs_n2 = a2[0:1] + a2[1:2]
    xs_norm = jnp.maximum(jnp.sqrt(xs_n2), jnp.float32(1e-8))

    kpad = jnp.concatenate([kv, jnp.zeros((7, 64), jnp.float32)], axis=0)
    dg = lax.dot_general(kpad.astype(jnp.bfloat16), xst.astype(jnp.bfloat16),
                         (((1,), (0,)), ((), ())),
                         preferred_element_type=jnp.float32)
    dots = dg[0:1, :]

    cos = dots / (xs_norm * k_norm)
    pos = (cos + jnp.float32(1.0)) / jnp.float32(2.0)
    gate = (count > 0).astype(jnp.float32)
    pos_ref[0] = pos * gate


def _topk_body(pos_ref, val_ref, idx_ref):
    v0 = pos_ref[...]

    def _treemax(ts):
        while len(ts) > 1:
            ts = [jnp.maximum(ts[2 * i], ts[2 * i + 1]) for i in range(len(ts) // 2)]
        return ts[0]

    def _treemin(ts):
        while len(ts) > 1:
            ts = [jnp.minimum(ts[2 * i], ts[2 * i + 1]) for i in range(len(ts) // 2)]
        return ts[0]

    def _half_step(v, iota):
        t = _treemax([v[:, 128 * g:128 * (g + 1)] for g in range(16)])
        mx = jnp.max(t, axis=1, keepdims=True)
        cand = jnp.where(v == mx, iota, jnp.int32(2048))
        tc = _treemin([cand[:, 128 * g:128 * (g + 1)] for g in range(16)])
        am = jnp.min(tc, axis=1, keepdims=True)
        vn = jnp.where(iota == am, -jnp.inf, v)
        return vn, mx, am

    iota16 = lax.broadcasted_iota(jnp.int32, (16, 2048), 1)
    riota = lax.broadcasted_iota(jnp.int32, (16, 256), 1)

    def body(r, carry):
        v, vals, idxs = carry
        v, mx, am = _half_step(v, iota16)
        sel = riota == r
        vals = jnp.where(sel, mx, vals)
        idxs = jnp.where(sel, am, idxs)
        return v, vals, idxs

    vals0 = jnp.zeros((16, 256), jnp.float32)
    idxs0 = jnp.zeros((16, 256), jnp.int32)
    _, vals, idxs = lax.fori_loop(0, 256, body, (v0, vals0, idxs0))
    val_ref[...] = vals
    idx_ref[...] = idxs


def _onehot_body(idx_ref, val_ref, out_ref):
    idr = idx_ref[0, 0]
    valr = val_ref[0, 0]
    idc = idr.reshape(64, 1)
    vc = valr.reshape(64, 1)
    iota = lax.broadcasted_iota(jnp.int32, (64, 2048), 1)
    oh = (iota == idc) & (vc > 0)
    out_ref[0] = oh.astype(jnp.float32)


def kernel(x_b, x_s, base_idxs):
    B, N, C = x_b.shape
    m = base_idxs.shape[1] // 2
    k = N // 8

    ids3 = base_idxs[:, :m].reshape(B, 1, m)

    pos3, ker3 = pl.pallas_call(
        _score_body,
        grid=(B,),
        in_specs=[
            pl.BlockSpec((1, 1, m), lambda b: (b, 0, 0),
                         memory_space=pltpu.SMEM),
            pl.BlockSpec((1, N, C), lambda b: (b, 0, 0)),
            pl.BlockSpec((1, N, C), lambda b: (b, 0, 0)),
        ],
        out_specs=[
            pl.BlockSpec((1, 1, N), lambda b: (b, 0, 0)),
            pl.BlockSpec((1, 1, C), lambda b: (b, 0, 0)),
        ],
        out_shape=[
            jax.ShapeDtypeStruct((B, 1, N), jnp.float32),
            jax.ShapeDtypeStruct((B, 1, C), jnp.float32),
        ],
    )(ids3, x_b, x_s)
    pos_scores = pos3.reshape(B, N)
    kernels = ker3.reshape(B, C)

    topk_val, topk_idx = pl.pallas_call(
        _topk_body,
        out_shape=[
            jax.ShapeDtypeStruct((B, k), jnp.float32),
            jax.ShapeDtypeStruct((B, k), jnp.int32),
        ],
    )(pos_scores)

    idx4 = topk_idx.reshape(B, 4, 1, 64)
    val4 = topk_val.reshape(B, 4, 1, 64)
    selected = pl.pallas_call(
        _onehot_body,
        grid=(B, 4),
        in_specs=[
            pl.BlockSpec((1, 1, 1, 64), lambda b, j: (b, j, 0, 0)),
            pl.BlockSpec((1, 1, 1, 64), lambda b, j: (b, j, 0, 0)),
        ],
        out_specs=pl.BlockSpec((1, 64, N), lambda b, j: (b, j, 0)),
        out_shape=jax.ShapeDtypeStruct((B, k, N), jnp.float32),
    )(idx4, val4)

    return (selected, topk_idx, pos_scores, x_s, kernels[:, :, None])


# speedup vs baseline: 1.6768x; 1.4170x over previous
import functools

import jax
import jax.numpy as jnp
from jax import lax
from jax.experimental import pallas as pl
from jax.experimental.pallas import tpu as pltpu


def _fold421_rows(rows):
    a4 = [rows[s] + rows[s + 4] for s in range(4)]
    a2 = [a4[s] + a4[s + 2] for s in range(2)]
    return a2[0] + a2[1]


def _lane64_reduce(s):
    acc = s[:, 0:8]
    for k in range(1, 8):
        acc = acc + s[:, 8 * k:8 * k + 8]
    a = acc[:, 0:4] + acc[:, 4:8]
    a = a[:, 0:2] + a[:, 2:4]
    return a[:, 0:1] + a[:, 1:2]


def _score_body(ids_ref, xb_ref, xs_ref, pos_ref, ker_ref):
    xs = xs_ref[0]

    count = jnp.zeros((), jnp.float32)
    parts = []
    for w0 in range(0, 512, 128):
        accs = [None] * 8
        for j in range(w0, w0 + 128, 8):
            for s in range(8):
                idx = ids_ref[0, 0, j + s]
                msk = (idx >= 0)
                idxc = jnp.maximum(idx, 0)
                row = xb_ref[0, pl.ds(idxc, 1), :] * jnp.where(msk, 1.0, 0.0).astype(jnp.float32)
                count = count + jnp.where(msk, 1.0, 0.0).astype(jnp.float32)
                accs[s] = row if accs[s] is None else accs[s] + row
        parts.append(_fold421_rows(accs))
    ksum = parts[0]
    for p in parts[1:]:
        ksum = ksum + p
    denom = jnp.maximum(count, jnp.float32(1.0))
    kv = ksum / denom
    ker_ref[0] = kv

    kn2 = _lane64_reduce(kv * kv)
    k_norm = jnp.maximum(jnp.sqrt(kn2), jnp.float32(1e-8))

    xst = xs.T
    sq = xst * xst
    accn = sq[0:8]
    for kk in range(1, 8):
        accn = accn + sq[8 * kk:8 * kk + 8]
    a4 = accn[0:4] + accn[4:8]
    a2 = a4[0:2] + a4[2:4]
    xs_n2 = a2[0:1] + a2[1:2]
    xs_norm = jnp.maximum(jnp.sqrt(xs_n2), jnp.float32(1e-8))

    kpad = jnp.concatenate([kv, jnp.zeros((7, 64), jnp.float32)], axis=0)
    dg = lax.dot_general(kpad.astype(jnp.bfloat16), xst.astype(jnp.bfloat16),
                         (((1,), (0,)), ((), ())),
                         preferred_element_type=jnp.float32)
    dots = dg[0:1, :]

    cos = dots / (xs_norm * k_norm)
    pos = (cos + jnp.float32(1.0)) / jnp.float32(2.0)
    gate = (count > 0).astype(jnp.float32)
    pos_ref[0] = pos * gate


def _topk_body(pos_ref, val_ref, idx_ref):
    v = pos_ref[...]
    N = 2048
    i = lax.broadcasted_iota(jnp.int32, (16, N), 1)
    j = lax.broadcasted_iota(jnp.int32, (16, N), 1)

    k = 2
    while k <= N:
        s = k // 2
        while s >= 1:
            pv_lo = pltpu.roll(v, N - s, 1)
            pv_hi = pltpu.roll(v, s, 1)
            pi_lo = pltpu.roll(i, N - s, 1)
            pi_hi = pltpu.roll(i, s, 1)
            side_hi = (j & s) != 0
            pv = jnp.where(side_hi, pv_hi, pv_lo)
            pi = jnp.where(side_hi, pi_hi, pi_lo)
            before = (v > pv) | ((v == pv) & (i < pi))
            asc = (j & k) != 0
            take_self = before ^ side_hi ^ asc
            v = jnp.where(take_self, v, pv)
            i = jnp.where(take_self, i, pi)
            s //= 2
        k *= 2

    val_ref[...] = v[:, 0:256]
    idx_ref[...] = i[:, 0:256]


def _onehot_body(idx_ref, val_ref, out_ref):
    idr = idx_ref[0, 0]
    valr = val_ref[0, 0]
    idc = idr.reshape(64, 1)
    vc = valr.reshape(64, 1)
    iota = lax.broadcasted_iota(jnp.int32, (64, 2048), 1)
    oh = (iota == idc) & (vc > 0)
    out_ref[0] = oh.astype(jnp.float32)


def kernel(x_b, x_s, base_idxs):
    B, N, C = x_b.shape
    m = base_idxs.shape[1] // 2
    k = N // 8

    ids3 = base_idxs[:, :m].reshape(B, 1, m)

    pos3, ker3 = pl.pallas_call(
        _score_body,
        grid=(B,),
        in_specs=[
            pl.BlockSpec((1, 1, m), lambda b: (b, 0, 0),
                         memory_space=pltpu.SMEM),
            pl.BlockSpec((1, N, C), lambda b: (b, 0, 0)),
            pl.BlockSpec((1, N, C), lambda b: (b, 0, 0)),
        ],
        out_specs=[
            pl.BlockSpec((1, 1, N), lambda b: (b, 0, 0)),
            pl.BlockSpec((1, 1, C), lambda b: (b, 0, 0)),
        ],
        out_shape=[
            jax.ShapeDtypeStruct((B, 1, N), jnp.float32),
            jax.ShapeDtypeStruct((B, 1, C), jnp.float32),
        ],
    )(ids3, x_b, x_s)
    pos_scores = pos3.reshape(B, N)
    kernels = ker3.reshape(B, C)

    topk_val, topk_idx = pl.pallas_call(
        _topk_body,
        out_shape=[
            jax.ShapeDtypeStruct((B, k), jnp.float32),
            jax.ShapeDtypeStruct((B, k), jnp.int32),
        ],
    )(pos_scores)

    idx4 = topk_idx.reshape(B, 4, 1, 64)
    val4 = topk_val.reshape(B, 4, 1, 64)
    selected = pl.pallas_call(
        _onehot_body,
        grid=(B, 4),
        in_specs=[
            pl.BlockSpec((1, 1, 1, 64), lambda b, j: (b, j, 0, 0)),
            pl.BlockSpec((1, 1, 1, 64), lambda b, j: (b, j, 0, 0)),
        ],
        out_specs=pl.BlockSpec((1, 64, N), lambda b, j: (b, j, 0)),
        out_shape=jax.ShapeDtypeStruct((B, k, N), jnp.float32),
    )(idx4, val4)

    return (selected, topk_idx, pos_scores, x_s, kernels[:, :, None])


# speedup vs baseline: 1.7600x; 1.0496x over previous
import functools

import jax
import jax.numpy as jnp
from jax import lax
from jax.experimental import pallas as pl
from jax.experimental.pallas import tpu as pltpu


def _fold421_rows(rows):
    a4 = [rows[s] + rows[s + 4] for s in range(4)]
    a2 = [a4[s] + a4[s + 2] for s in range(2)]
    return a2[0] + a2[1]


def _lane64_reduce(s):
    acc = s[:, 0:8]
    for k in range(1, 8):
        acc = acc + s[:, 8 * k:8 * k + 8]
    a = acc[:, 0:4] + acc[:, 4:8]
    a = a[:, 0:2] + a[:, 2:4]
    return a[:, 0:1] + a[:, 1:2]


def _score_body(ids_ref, xb_ref, xs_ref, pos_ref, ker_ref):
    xs = xs_ref[0]

    count = jnp.zeros((), jnp.float32)
    parts = []
    for w0 in range(0, 512, 128):
        accs = [None] * 8
        for j in range(w0, w0 + 128, 8):
            for s in range(8):
                idx = ids_ref[0, 0, j + s]
                msk = (idx >= 0)
                idxc = jnp.maximum(idx, 0)
                row = xb_ref[0, pl.ds(idxc, 1), :] * jnp.where(msk, 1.0, 0.0).astype(jnp.float32)
                count = count + jnp.where(msk, 1.0, 0.0).astype(jnp.float32)
                accs[s] = row if accs[s] is None else accs[s] + row
        parts.append(_fold421_rows(accs))
    ksum = parts[0]
    for p in parts[1:]:
        ksum = ksum + p
    denom = jnp.maximum(count, jnp.float32(1.0))
    kv = ksum / denom
    ker_ref[0] = kv

    kn2 = _lane64_reduce(kv * kv)
    k_norm = jnp.maximum(jnp.sqrt(kn2), jnp.float32(1e-8))

    xst = xs.T
    sq = xst * xst
    accn = sq[0:8]
    for kk in range(1, 8):
        accn = accn + sq[8 * kk:8 * kk + 8]
    a4 = accn[0:4] + accn[4:8]
    a2 = a4[0:2] + a4[2:4]
    xs_n2 = a2[0:1] + a2[1:2]
    xs_norm = jnp.maximum(jnp.sqrt(xs_n2), jnp.float32(1e-8))

    kpad = jnp.concatenate([kv, jnp.zeros((7, 64), jnp.float32)], axis=0)
    dg = lax.dot_general(kpad.astype(jnp.bfloat16), xst.astype(jnp.bfloat16),
                         (((1,), (0,)), ((), ())),
                         preferred_element_type=jnp.float32)
    dots = dg[0:1, :]

    cos = dots / (xs_norm * k_norm)
    pos = (cos + jnp.float32(1.0)) / jnp.float32(2.0)
    gate = (count > 0).astype(jnp.float32)
    pos_ref[0] = pos * gate


def _topk_body(pos_ref, val_ref, idx_ref):
    pos = pos_ref[...]
    A = pos.reshape(16, 8, 256).transpose(2, 1, 0).reshape(256, 128)
    lane = lax.broadcasted_iota(jnp.int32, (256, 128), 1)
    p = lax.broadcasted_iota(jnp.int32, (256, 128), 0)
    c = lane >> 4
    I = c * 256 + p

    def ebit(mask):
        if mask < 256:
            return (p & mask) != 0
        return (c & (mask >> 8)) != 0

    def partner(x, s):
        if s < 8:
            return jnp.where((p & s) != 0,
                             pltpu.roll(x, s, 0), pltpu.roll(x, 256 - s, 0))
        if s < 256:
            segs = []
            for b0 in range(0, 256, 2 * s):
                segs.append(x[b0 + s:b0 + 2 * s])
                segs.append(x[b0:b0 + s])
            return jnp.concatenate(segs, axis=0)
        g16 = (s >> 8) * 16
        return jnp.where((lane & g16) != 0,
                         pltpu.roll(x, g16, 1), pltpu.roll(x, 128 - g16, 1))

    k = 2
    while k <= 2048:
        s = k // 2
        while s >= 1:
            pv = partner(A, s)
            pi = partner(I, s)
            side_hi = ebit(s)
            before = (A > pv) | ((A == pv) & (I < pi))
            asc = ebit(k)
            take_self = before ^ side_hi ^ asc
            A = jnp.where(take_self, A, pv)
            I = jnp.where(take_self, I, pi)
            s //= 2
        k *= 2

    val_ref[...] = A[:, 0:16].T
    idx_ref[...] = I[:, 0:16].T


def _onehot_body(idx_ref, val_ref, out_ref):
    idr = idx_ref[0, 0]
    valr = val_ref[0, 0]
    idc = idr.reshape(64, 1)
    vc = valr.reshape(64, 1)
    iota = lax.broadcasted_iota(jnp.int32, (64, 2048), 1)
    oh = (iota == idc) & (vc > 0)
    out_ref[0] = oh.astype(jnp.float32)


def kernel(x_b, x_s, base_idxs):
    B, N, C = x_b.shape
    m = base_idxs.shape[1] // 2
    k = N // 8

    ids3 = base_idxs[:, :m].reshape(B, 1, m)

    pos3, ker3 = pl.pallas_call(
        _score_body,
        grid=(B,),
        in_specs=[
            pl.BlockSpec((1, 1, m), lambda b: (b, 0, 0),
                         memory_space=pltpu.SMEM),
            pl.BlockSpec((1, N, C), lambda b: (b, 0, 0)),
            pl.BlockSpec((1, N, C), lambda b: (b, 0, 0)),
        ],
        out_specs=[
            pl.BlockSpec((1, 1, N), lambda b: (b, 0, 0)),
            pl.BlockSpec((1, 1, C), lambda b: (b, 0, 0)),
        ],
        out_shape=[
            jax.ShapeDtypeStruct((B, 1, N), jnp.float32),
            jax.ShapeDtypeStruct((B, 1, C), jnp.float32),
        ],
    )(ids3, x_b, x_s)
    pos_scores = pos3.reshape(B, N)
    kernels = ker3.reshape(B, C)

    topk_val, topk_idx = pl.pallas_call(
        _topk_body,
        out_shape=[
            jax.ShapeDtypeStruct((B, k), jnp.float32),
            jax.ShapeDtypeStruct((B, k), jnp.int32),
        ],
    )(pos_scores)

    idx4 = topk_idx.reshape(B, 4, 1, 64)
    val4 = topk_val.reshape(B, 4, 1, 64)
    selected = pl.pallas_call(
        _onehot_body,
        grid=(B, 4),
        in_specs=[
            pl.BlockSpec((1, 1, 1, 64), lambda b, j: (b, j, 0, 0)),
            pl.BlockSpec((1, 1, 1, 64), lambda b, j: (b, j, 0, 0)),
        ],
        out_specs=pl.BlockSpec((1, 64, N), lambda b, j: (b, j, 0)),
        out_shape=jax.ShapeDtypeStruct((B, k, N), jnp.float32),
    )(idx4, val4)

    return (selected, topk_idx, pos_scores, x_s, kernels[:, :, None])


# speedup vs baseline: 2.2100x; 1.2557x over previous
import functools

import jax
import jax.numpy as jnp
from jax import lax
from jax.experimental import pallas as pl
from jax.experimental.pallas import tpu as pltpu


def _fold421_rows(rows):
    a4 = [rows[s] + rows[s + 4] for s in range(4)]
    a2 = [a4[s] + a4[s + 2] for s in range(2)]
    return a2[0] + a2[1]


def _lane64_reduce(s):
    acc = s[:, 0:8]
    for k in range(1, 8):
        acc = acc + s[:, 8 * k:8 * k + 8]
    a = acc[:, 0:4] + acc[:, 4:8]
    a = a[:, 0:2] + a[:, 2:4]
    return a[:, 0:1] + a[:, 1:2]


def _score_body(ids_ref, xbt_ref, xst_ref, pos_ref, ker_ref, xb_ref):
    xb_ref[...] = xbt_ref[0].T

    count = jnp.zeros((), jnp.float32)
    parts = []
    for w0 in range(0, 512, 128):
        accs = [None] * 8
        for j in range(w0, w0 + 128, 8):
            for s in range(8):
                idx = ids_ref[0, 0, j + s]
                msk = (idx >= 0)
                idxc = jnp.maximum(idx, 0)
                row = xb_ref[pl.ds(idxc, 1), :] * jnp.where(msk, 1.0, 0.0).astype(jnp.float32)
                count = count + jnp.where(msk, 1.0, 0.0).astype(jnp.float32)
                accs[s] = row if accs[s] is None else accs[s] + row
        parts.append(_fold421_rows(accs))
    ksum = parts[0]
    for p in parts[1:]:
        ksum = ksum + p
    denom = jnp.maximum(count, jnp.float32(1.0))
    kv = ksum / denom
    ker_ref[0] = kv

    kn2 = _lane64_reduce(kv * kv)
    k_norm = jnp.maximum(jnp.sqrt(kn2), jnp.float32(1e-8))

    xst = xst_ref[0]
    sq = xst * xst
    accn = sq[0:8]
    for kk in range(1, 8):
        accn = accn + sq[8 * kk:8 * kk + 8]
    a4 = accn[0:4] + accn[4:8]
    a2 = a4[0:2] + a4[2:4]
    xs_n2 = a2[0:1] + a2[1:2]
    xs_norm = jnp.maximum(jnp.sqrt(xs_n2), jnp.float32(1e-8))

    kpad = jnp.concatenate([kv, jnp.zeros((7, 64), jnp.float32)], axis=0)
    dg = lax.dot_general(kpad.astype(jnp.bfloat16), xst.astype(jnp.bfloat16),
                         (((1,), (0,)), ((), ())),
                         preferred_element_type=jnp.float32)
    dots = dg[0:1, :]

    cos = dots / (xs_norm * k_norm)
    pos = (cos + jnp.float32(1.0)) / jnp.float32(2.0)
    gate = (count > 0).astype(jnp.float32)
    pos_ref[0] = pos * gate


def _topk_body(pos_ref, val_ref, idx_ref):
    pos = pos_ref[...]
    A = pos.reshape(16, 8, 256).transpose(2, 1, 0).reshape(256, 128)
    lane = lax.broadcasted_iota(jnp.int32, (256, 128), 1)
    p = lax.broadcasted_iota(jnp.int32, (256, 128), 0)
    c = lane >> 4
    I = c * 256 + p

    def ebit(mask):
        if mask < 256:
            return (p & mask) != 0
        return (c & (mask >> 8)) != 0

    def partner(x, s):
        if s < 8:
            return jnp.where((p & s) != 0,
                             pltpu.roll(x, s, 0), pltpu.roll(x, 256 - s, 0))
        if s < 256:
            segs = []
            for b0 in range(0, 256, 2 * s):
                segs.append(x[b0 + s:b0 + 2 * s])
                segs.append(x[b0:b0 + s])
            return jnp.concatenate(segs, axis=0)
        g16 = (s >> 8) * 16
        return jnp.where((lane & g16) != 0,
                         pltpu.roll(x, g16, 1), pltpu.roll(x, 128 - g16, 1))

    k = 2
    while k <= 2048:
        s = k // 2
        while s >= 1:
            pv = partner(A, s)
            pi = partner(I, s)
            side_hi = ebit(s)
            before = (A > pv) | ((A == pv) & (I < pi))
            asc = ebit(k)
            take_self = before ^ side_hi ^ asc
            A = jnp.where(take_self, A, pv)
            I = jnp.where(take_self, I, pi)
            s //= 2
        k *= 2

    val_ref[...] = A[:, 0:16].T
    idx_ref[...] = I[:, 0:16].T


def _onehot_body(idx_ref, val_ref, out_ref):
    idr = idx_ref[0, 0]
    valr = val_ref[0, 0]
    idc = idr.reshape(64, 1)
    vc = valr.reshape(64, 1)
    iota = lax.broadcasted_iota(jnp.int32, (64, 2048), 1)
    oh = (iota == idc) & (vc > 0)
    out_ref[0] = oh.astype(jnp.float32)


def kernel(x_b, x_s, base_idxs):
    B, N, C = x_b.shape
    m = base_idxs.shape[1] // 2
    k = N // 8

    ids3 = base_idxs[:, :m].reshape(B, 1, m)
    x_bt = jnp.swapaxes(x_b, 1, 2)
    x_st = jnp.swapaxes(x_s, 1, 2)

    pos3, ker3 = pl.pallas_call(
        _score_body,
        grid=(B,),
        in_specs=[
            pl.BlockSpec((1, 1, m), lambda b: (b, 0, 0),
                         memory_space=pltpu.SMEM),
            pl.BlockSpec((1, C, N), lambda b: (b, 0, 0)),
            pl.BlockSpec((1, C, N), lambda b: (b, 0, 0)),
        ],
        out_specs=[
            pl.BlockSpec((1, 1, N), lambda b: (b, 0, 0)),
            pl.BlockSpec((1, 1, C), lambda b: (b, 0, 0)),
        ],
        out_shape=[
            jax.ShapeDtypeStruct((B, 1, N), jnp.float32),
            jax.ShapeDtypeStruct((B, 1, C), jnp.float32),
        ],
        scratch_shapes=[pltpu.VMEM((N, C), jnp.float32)],
    )(ids3, x_bt, x_st)
    pos_scores = pos3.reshape(B, N)
    kernels = ker3.reshape(B, C)

    topk_val, topk_idx = pl.pallas_call(
        _topk_body,
        out_shape=[
            jax.ShapeDtypeStruct((B, k), jnp.float32),
            jax.ShapeDtypeStruct((B, k), jnp.int32),
        ],
    )(pos_scores)

    idx4 = topk_idx.reshape(B, 4, 1, 64)
    val4 = topk_val.reshape(B, 4, 1, 64)
    selected = pl.pallas_call(
        _onehot_body,
        grid=(B, 4),
        in_specs=[
            pl.BlockSpec((1, 1, 1, 64), lambda b, j: (b, j, 0, 0)),
            pl.BlockSpec((1, 1, 1, 64), lambda b, j: (b, j, 0, 0)),
        ],
        out_specs=pl.BlockSpec((1, 64, N), lambda b, j: (b, j, 0)),
        out_shape=jax.ShapeDtypeStruct((B, k, N), jnp.float32),
    )(idx4, val4)

    return (selected, topk_idx, pos_scores, x_s, kernels[:, :, None])
